# baseline (speedup 1.0000x reference)
import functools

import jax
import jax.numpy as jnp
from jax import lax
from jax.experimental import pallas as pl
from jax.experimental.pallas import tpu as pltpu
from jax.experimental.pallas import tpu_sc as plsc

_NC = 2
_NS = 16
_NW = _NC * _NS
_G = 128


def _dot(a, b):
    return lax.dot_general(a, b, (((1,), (0,)), ((), ())),
                           precision=lax.Precision.HIGHEST,
                           preferred_element_type=jnp.float32)


def _mesh():
    return plsc.VectorSubcoreMesh(core_axis_name="c", subcore_axis_name="s",
                                  num_cores=_NC, num_subcores=_NS)


_K = 1


def _make_sc_gather(npad, epad, ew, width):
    grp = _K * _G
    ngt = ew // grp
    ngk = epad // grp

    @functools.partial(
        pl.kernel,
        out_type=jax.ShapeDtypeStruct((ngk, 2 * grp, width), jnp.float32),
        mesh=_mesh(),
        scratch_types=[
            pltpu.VMEM((grp,), jnp.int32),
            pltpu.VMEM((grp,), jnp.int32),
            pltpu.VMEM((grp, width), jnp.float32),
            pltpu.VMEM((grp, width), jnp.float32),
            pltpu.SemaphoreType.DMA,
        ],
    )
    def k(pq_hbm, pq2_hbm, comb_hbm, out_hbm, idxd, idxs, bufp, bufq, sem):
        cid = lax.axis_index("c")
        sid = lax.axis_index("s")
        wid = sid * _NC + cid
        g0 = wid * ngt

        def body(g, carry):
            b = (g0 + g) * 2 * grp
            pltpu.sync_copy(comb_hbm.at[pl.ds(b, grp)], idxd)
            pltpu.sync_copy(comb_hbm.at[pl.ds(b + grp, grp)], idxs)
            ca = pltpu.async_copy(pq_hbm.at[idxd], bufp, sem)
            cb = pltpu.async_copy(pq2_hbm.at[idxs], bufq, sem)
            ca.wait()
            cb.wait()
            pltpu.sync_copy(bufp, out_hbm.at[g0 + g, pl.ds(0, grp)])
            pltpu.sync_copy(bufq, out_hbm.at[g0 + g, pl.ds(grp, grp)])
            return carry

        lax.fori_loop(0, ngt, body, 0)

    return k


def _make_sc_scatter(npad, epad, ew, width):
    ng = ew // _G
    stripe = npad // _NS
    nz = stripe // _G

    @functools.partial(
        pl.kernel,
        out_type=jax.ShapeDtypeStruct((_NC, npad, width), jnp.float32),
        mesh=_mesh(),
        scratch_types=[
            pltpu.VMEM((_G,), jnp.int32),
            pltpu.VMEM((_G, width), jnp.float32),
            pltpu.VMEM_SHARED((npad, width), jnp.float32),
        ],
    )
    def k(m_hbm, dst_hbm, zeros_hbm, out_hbm, idx, mbuf, table):
        cid = lax.axis_index("c")
        sid = lax.axis_index("s")
        wid = sid * _NC + cid
        base0 = wid * ew

        def zero_body(i, carry):
            pltpu.sync_copy(zeros_hbm,
                            table.at[pl.ds(sid * stripe + i * _G, _G)])
            return carry

        lax.fori_loop(0, nz, zero_body, 0)
        plsc.subcore_barrier()

        def body(g, carry):
            b = base0 + g * _G
            pltpu.sync_copy(dst_hbm.at[pl.ds(b, _G)], idx)
            pltpu.sync_copy(m_hbm.at[pl.ds(b, _G)], mbuf)
            pltpu.sync_copy(mbuf, table.at[idx], add=True)
            return carry

        lax.fori_loop(0, ng, body, 0)
        plsc.subcore_barrier()

        def wb_body(i, carry):
            r = sid * stripe + i * _G
            pltpu.sync_copy(table.at[pl.ds(r, _G)], mbuf)
            pltpu.sync_copy(mbuf, out_hbm.at[cid, pl.ds(r, _G)])
            return carry

        lax.fori_loop(0, nz, wb_body, 0)

    return k


def _full(shape):
    return pl.BlockSpec(shape, lambda i: tuple(0 for _ in shape))


def _enc_body(rv_ref, e8_ref, ce_ref, w2_ref, b2_ref, wd_ref, ws_ref, wv_ref,
              h_ref, pq_ref):
    rv = rv_ref[...]
    t = jnp.maximum(_dot(rv, e8_ref[...]) + ce_ref[...], 0.0)
    h = jnp.maximum(_dot(t, w2_ref[...]) + b2_ref[...], 0.0)
    h_ref[...] = h
    vterm = _dot(rv, wv_ref[...])
    pq_ref[0] = _dot(h, wd_ref[...]) + vterm
    pq_ref[1] = _dot(h, ws_ref[...]) - vterm


def _halves(s2_ref):
    blk = s2_ref[...]
    bg, g2, hid = blk.shape
    rows = bg * (g2 // 2)
    a = blk[:, :g2 // 2].reshape(rows, hid)
    b = blk[:, g2 // 2:].reshape(rows, hid)
    return a, b


def _dx_body(d2_ref, dom_ref, dx_ref):
    a, b = _halves(d2_ref)
    d = (a - b)[:, :16]
    dom = dom_ref[...]
    dx_ref[...] = d - dom * jnp.round(d / dom)


def _edge_body(s2_ref, dx_ref, wx_ref, cm_ref, w2_ref, b2_ref, m_ref):
    a, b = _halves(s2_ref)
    pre = a + b + _dot(dx_ref[...], wx_ref[...]) + cm_ref[...]
    t = jnp.maximum(pre, 0.0)
    m_ref[...] = jnp.maximum(_dot(t, w2_ref[...]) + b2_ref[...], 0.0)


def _upd_common(h_ref, agg_ref, u1a_ref, u1b_ref, cu_ref, u2_ref, bu2_ref,
                lns_ref, lnb_ref):
    h = h_ref[...]
    agg = agg_ref[0] + agg_ref[1]
    t = jnp.maximum(_dot(h, u1a_ref[...]) + _dot(agg, u1b_ref[...]) + cu_ref[...], 0.0)
    o = h + _dot(t, u2_ref[...]) + bu2_ref[...]
    mu = jnp.mean(o, axis=1, keepdims=True)
    d = o - mu
    var = jnp.mean(d * d, axis=1, keepdims=True)
    return d * lax.rsqrt(var + 1e-5) * lns_ref[...] + lnb_ref[...]


def _upd_mid_body(h_ref, agg_ref, u1a_ref, u1b_ref, cu_ref, u2_ref, bu2_ref,
                  lns_ref, lnb_ref, rv_ref, wd_ref, ws_ref, wv_ref,
                  hn_ref, pq_ref):
    hn = _upd_common(h_ref, agg_ref, u1a_ref, u1b_ref, cu_ref, u2_ref, bu2_ref,
                     lns_ref, lnb_ref)
    hn_ref[...] = hn
    vterm = _dot(rv_ref[...], wv_ref[...])
    pq_ref[0] = _dot(hn, wd_ref[...]) + vterm
    pq_ref[1] = _dot(hn, ws_ref[...]) - vterm


def _upd_last_body(h_ref, agg_ref, u1a_ref, u1b_ref, cu_ref, u2_ref, bu2_ref,
                   lns_ref, lnb_ref, hn_ref):
    hn_ref[...] = _upd_common(h_ref, agg_ref, u1a_ref, u1b_ref, cu_ref, u2_ref,
                              bu2_ref, lns_ref, lnb_ref)


def _dec_body(h_ref, o1_ref, bo1_ref, o2_ref, bo2_ref, nf_ref, dom_ref,
              pred_ref, px_ref):
    t = jnp.maximum(_dot(h_ref[...], o1_ref[...]) + bo1_ref[...], 0.0)
    pred = _dot(t, o2_ref[...]) + bo2_ref[...] + nf_ref[...]
    pred_ref[...] = pred
    dom = dom_ref[...]
    px_ref[...] = pred - dom * jnp.floor(pred / dom)


def _pool_body(n_real, bn, h_ref, o_ref):
    i = pl.program_id(0)

    @pl.when(i == 0)
    def _():
        o_ref[...] = jnp.zeros_like(o_ref)

    rows = lax.broadcasted_iota(jnp.int32, (bn, 1), 0) + i * bn
    hm = jnp.where(rows < n_real, h_ref[...], 0.0)
    o_ref[...] += jnp.sum(hm, axis=0, keepdims=True)


def _macro_body(n_real, s_ref, m1_ref, bm1_ref, m2_ref, bm2_ref, o_ref):
    hm = s_ref[...] * (1.0 / n_real)
    t = jnp.maximum(_dot(hm, m1_ref[...]) + bm1_ref[...], 0.0)
    o_ref[...] = _dot(t, m2_ref[...]) + bm2_ref[...]


def kernel(v, pos, r, domain, t, x_global, domain_next, t_next, edge_index,
           batch, params):
    f32 = jnp.float32
    n = v.shape[0]
    e = edge_index.shape[1]
    hid = params['emb2'][0].shape[1]

    npad = -(-n // (_NS * _G)) * (_NS * _G)
    ew = -(-e // (_NW * _G * 6)) * _G * 6
    epad = ew * _NW

    gfeat = jnp.concatenate([domain, t, x_global, domain_next, t_next])

    src = edge_index[0]
    dst = edge_index[1]
    pad_idx = jnp.full((epad - e,), npad - 1, jnp.int32)
    srcp = jnp.concatenate([src, pad_idx])
    dstp = jnp.concatenate([dst, pad_idx])
    grp = _K * _G
    ngk = epad // grp
    ngt = ew // grp
    dst3 = dstp.reshape(ngk, grp)
    src3 = srcp.reshape(ngk, grp) + npad
    comb = jnp.concatenate([dst3, src3], axis=1).reshape(-1)
    dst4 = dstp

    def padrows(x, rows, cols):
        return jnp.pad(x, ((0, rows - x.shape[0]), (0, cols - x.shape[1])))

    rv = padrows(jnp.concatenate([r, v], axis=1), npad, 8)
    pos128 = padrows(pos, npad, hid)
    nf16 = padrows(jnp.concatenate([pos, v], axis=1), npad, 16)
    dom16 = jnp.concatenate([domain, jnp.ones((13,), f32)]).reshape(1, 16)
    domn16 = jnp.concatenate([domain_next, jnp.ones((13,), f32)]).reshape(1, 16)
    zeros128 = jnp.zeros((_G, hid), f32)

    e1, be1 = params['emb1']
    e8 = jnp.pad(e1[:7], ((0, 1), (0, 0)))
    ce = (gfeat @ e1[7:19] + be1).reshape(1, hid)
    e2, be2 = params['emb2']
    be2 = be2.reshape(1, hid)

    lw = []
    for lp in params['layers']:
        w1, b1 = lp['msg1']
        wd = w1[0:hid]
        ws = w1[hid:2 * hid]
        wx16 = jnp.pad(w1[2 * hid:2 * hid + 3], ((0, 13), (0, 0)))
        wv8 = jnp.pad(w1[2 * hid + 3:2 * hid + 9], ((1, 1), (0, 0)))
        cm = (gfeat @ w1[2 * hid + 9:] + b1).reshape(1, hid)
        w2, b2 = lp['msg2']
        u1, bu1 = lp['upd1']
        u1a = u1[0:hid]
        u1b = u1[hid:2 * hid]
        cu = (gfeat @ u1[2 * hid:] + bu1).reshape(1, hid)
        u2, bu2 = lp['upd2']
        lw.append(dict(wd=wd, ws=ws, wx16=wx16, wv8=wv8, cm=cm, w2=w2,
                       b2=b2.reshape(1, hid), u1a=u1a, u1b=u1b, cu=cu, u2=u2,
                       bu2=bu2.reshape(1, hid), lns=lp['ln_s'].reshape(1, hid),
                       lnb=lp['ln_b'].reshape(1, hid)))

    o1, bo1 = params['out1']
    o2, bo2 = params['out2']
    o2p = jnp.pad(o2, ((0, 0), (0, 16 - o2.shape[1])))
    bo2p = jnp.pad(bo2, (0, 16 - bo2.shape[0])).reshape(1, 16)
    m1, bm1 = params['mac1']
    m2, bm2 = params['mac2']
    m2p = jnp.pad(m2, ((0, 0), (0, 16 - m2.shape[1])))
    bm2p = jnp.pad(bm2, (0, 16 - bm2.shape[0])).reshape(1, 16)

    bn = 512
    gn = npad // bn
    be = 512
    ge = epad // be

    w128 = _full((hid, hid))
    w1h = _full((1, hid))

    bg = be // grp
    pq_spec = pl.BlockSpec((2, bn, hid), lambda i: (0, i, 0))
    pq_shape = jax.ShapeDtypeStruct((2, npad, hid), f32)
    s2_spec = pl.BlockSpec((bg, 2 * grp, hid), lambda i: (i, 0, 0))

    enc = pl.pallas_call(
        _enc_body,
        grid=(gn,),
        in_specs=[pl.BlockSpec((bn, 8), lambda i: (i, 0)), _full((8, hid)),
                  w1h, w128, w1h, w128, w128, _full((8, hid))],
        out_specs=[pl.BlockSpec((bn, hid), lambda i: (i, 0)), pq_spec],
        out_shape=[jax.ShapeDtypeStruct((npad, hid), f32), pq_shape],
    )

    dxk = pl.pallas_call(
        _dx_body,
        grid=(ge,),
        in_specs=[s2_spec, _full((1, 16))],
        out_specs=pl.BlockSpec((be, 16), lambda i: (i, 0)),
        out_shape=jax.ShapeDtypeStruct((epad, 16), f32),
    )

    edgek = pl.pallas_call(
        _edge_body,
        grid=(ge,),
        in_specs=[s2_spec,
                  pl.BlockSpec((be, 16), lambda i: (i, 0)),
                  _full((16, hid)), w1h, w128, w1h],
        out_specs=pl.BlockSpec((be, hid), lambda i: (i, 0)),
        out_shape=jax.ShapeDtypeStruct((epad, hid), f32),
    )

    upd_mid = pl.pallas_call(
        _upd_mid_body,
        grid=(gn,),
        in_specs=[pl.BlockSpec((bn, hid), lambda i: (i, 0)),
                  pl.BlockSpec((2, bn, hid), lambda i: (0, i, 0)),
                  w128, w128, w1h, w128, w1h, w1h, w1h,
                  pl.BlockSpec((bn, 8), lambda i: (i, 0)),
                  w128, w128, _full((8, hid))],
        out_specs=[pl.BlockSpec((bn, hid), lambda i: (i, 0)), pq_spec],
        out_shape=[jax.ShapeDtypeStruct((npad, hid), f32), pq_shape],
    )

    upd_last = pl.pallas_call(
        _upd_last_body,
        grid=(gn,),
        in_specs=[pl.BlockSpec((bn, hid), lambda i: (i, 0)),
                  pl.BlockSpec((2, bn, hid), lambda i: (0, i, 0)),
                  w128, w128, w1h, w128, w1h, w1h, w1h],
        out_specs=pl.BlockSpec((bn, hid), lambda i: (i, 0)),
        out_shape=jax.ShapeDtypeStruct((npad, hid), f32),
    )

    deck = pl.pallas_call(
        _dec_body,
        grid=(gn,),
        in_specs=[pl.BlockSpec((bn, hid), lambda i: (i, 0)),
                  w128, w1h, _full((hid, 16)), _full((1, 16)),
                  pl.BlockSpec((bn, 16), lambda i: (i, 0)), _full((1, 16))],
        out_specs=[pl.BlockSpec((bn, 16), lambda i: (i, 0))] * 2,
        out_shape=[jax.ShapeDtypeStruct((npad, 16), f32)] * 2,
    )

    poolk = pl.pallas_call(
        functools.partial(_pool_body, n, bn),
        grid=(gn,),
        in_specs=[pl.BlockSpec((bn, hid), lambda i: (i, 0))],
        out_specs=pl.BlockSpec((1, hid), lambda i: (0, 0)),
        out_shape=jax.ShapeDtypeStruct((1, hid), f32),
    )

    macrok = pl.pallas_call(
        functools.partial(_macro_body, float(n)),
        grid=(1,),
        in_specs=[_full((1, hid)), w128, w1h, _full((hid, 16)), _full((1, 16))],
        out_specs=_full((1, 16)),
        out_shape=jax.ShapeDtypeStruct((1, 16), f32),
    )

    sc_gather = _make_sc_gather(npad, epad, ew, hid)
    sc_scatter = _make_sc_scatter(npad, epad, ew, hid)

    h, pq = enc(rv, e8, ce, e2, be2, lw[0]['wd'], lw[0]['ws'], lw[0]['wv8'])

    pos2 = jnp.concatenate([pos128, pos128], axis=0)
    d2 = sc_gather(pos2, pos2 + 0.0, comb)
    dxw = dxk(d2, dom16)

    nl = len(lw)
    for l, w in enumerate(lw):
        pq2 = pq.reshape(2 * npad, hid)
        s2 = sc_gather(pq2, pq2 + 0.0, comb)
        m = edgek(s2, dxw, w['wx16'], w['cm'], w['w2'], w['b2'])
        agg2 = sc_scatter(m, dst4, zeros128)
        if l + 1 < nl:
            wn = lw[l + 1]
            h, pq = upd_mid(h, agg2, w['u1a'], w['u1b'], w['cu'], w['u2'],
                            w['bu2'], w['lns'], w['lnb'], rv,
                            wn['wd'], wn['ws'], wn['wv8'])
        else:
            h = upd_last(h, agg2, w['u1a'], w['u1b'], w['cu'], w['u2'],
                         w['bu2'], w['lns'], w['lnb'])

    pred16, px16 = deck(h, o1, bo1.reshape(1, hid), o2p, bo2p, nf16, domn16)
    hsum = poolk(h)
    macro16 = macrok(hsum, m1, bm1.reshape(1, hid), m2p, bm2p)

    pred_x = px16[:n, 0:3]
    pred_v = pred16[:n, 3:9]
    pred_macro = macro16[0, :3]
    return (pred_x, pred_v, pred_macro)


# speedup vs baseline: 1.5395x; 1.5395x over previous
import functools

import jax
import jax.numpy as jnp
from jax import lax
from jax.experimental import pallas as pl
from jax.experimental.pallas import tpu as pltpu
from jax.experimental.pallas import tpu_sc as plsc

_NC = 2
_NS = 16
_NW = _NC * _NS
_G = 128


def _dot(a, b):
    return lax.dot_general(a, b, (((1,), (0,)), ((), ())),
                           precision=lax.Precision.HIGHEST,
                           preferred_element_type=jnp.float32)


def _mesh():
    return plsc.VectorSubcoreMesh(core_axis_name="c", subcore_axis_name="s",
                                  num_cores=_NC, num_subcores=_NS)


_K = 1


def _make_sc_gather(npad, epad, ew, width):
    ng = ew // _G

    @functools.partial(
        pl.kernel,
        out_type=jax.ShapeDtypeStruct((2, epad, width), jnp.float32),
        mesh=_mesh(),
        scratch_types=[
            pltpu.VMEM((_G,), jnp.int32),
            pltpu.VMEM((_G,), jnp.int32),
            pltpu.VMEM((_G, width), jnp.float32),
            pltpu.VMEM((_G, width), jnp.float32),
            pltpu.SemaphoreType.DMA,
        ],
    )
    def k(p_hbm, q_hbm, dst_hbm, src_hbm, out_hbm, idxd, idxs, bufp, bufq,
          sem):
        cid = lax.axis_index("c")
        sid = lax.axis_index("s")
        wid = sid * _NC + cid
        base0 = wid * ew

        def body(g, carry):
            b = base0 + g * _G
            pltpu.sync_copy(dst_hbm.at[pl.ds(b, _G)], idxd)
            pltpu.sync_copy(src_hbm.at[pl.ds(b, _G)], idxs)
            ca = pltpu.async_copy(p_hbm.at[idxd], bufp, sem)
            cb = pltpu.async_copy(q_hbm.at[idxs], bufq, sem)
            ca.wait()
            cb.wait()
            pltpu.sync_copy(bufp, out_hbm.at[0, pl.ds(b, _G)])
            pltpu.sync_copy(bufq, out_hbm.at[1, pl.ds(b, _G)])
            return carry

        lax.fori_loop(0, ng, body, 0)

    return k


def _make_sc_scatter(npad, epad, ew, width):
    ng = ew // _G
    stripe = npad // _NS
    nz = stripe // _G

    @functools.partial(
        pl.kernel,
        out_type=jax.ShapeDtypeStruct((_NC, npad, width), jnp.float32),
        mesh=_mesh(),
        scratch_types=[
            pltpu.VMEM((_G,), jnp.int32),
            pltpu.VMEM((_G, width), jnp.float32),
            pltpu.VMEM_SHARED((npad, width), jnp.float32),
        ],
    )
    def k(m_hbm, dst_hbm, zeros_hbm, out_hbm, idx, mbuf, table):
        cid = lax.axis_index("c")
        sid = lax.axis_index("s")
        wid = sid * _NC + cid
        base0 = wid * ew

        def zero_body(i, carry):
            pltpu.sync_copy(zeros_hbm,
                            table.at[pl.ds(sid * stripe + i * _G, _G)])
            return carry

        lax.fori_loop(0, nz, zero_body, 0)
        plsc.subcore_barrier()

        def body(g, carry):
            b = base0 + g * _G
            pltpu.sync_copy(dst_hbm.at[pl.ds(b, _G)], idx)
            pltpu.sync_copy(m_hbm.at[pl.ds(b, _G)], mbuf)
            pltpu.sync_copy(mbuf, table.at[idx], add=True)
            return carry

        lax.fori_loop(0, ng, body, 0)
        plsc.subcore_barrier()

        def wb_body(i, carry):
            r = sid * stripe + i * _G
            pltpu.sync_copy(table.at[pl.ds(r, _G)], mbuf)
            pltpu.sync_copy(mbuf, out_hbm.at[cid, pl.ds(r, _G)])
            return carry

        lax.fori_loop(0, nz, wb_body, 0)

    return k


def _full(shape):
    return pl.BlockSpec(shape, lambda i: tuple(0 for _ in shape))


def _enc_body(rv_ref, e8_ref, ce_ref, w2_ref, b2_ref, wd_ref, ws_ref, wv_ref,
              h_ref, p_ref, q_ref):
    rv = rv_ref[...]
    t = jnp.maximum(_dot(rv, e8_ref[...]) + ce_ref[...], 0.0)
    h = jnp.maximum(_dot(t, w2_ref[...]) + b2_ref[...], 0.0)
    h_ref[...] = h
    vterm = _dot(rv, wv_ref[...])
    p_ref[...] = _dot(h, wd_ref[...]) + vterm
    q_ref[...] = _dot(h, ws_ref[...]) - vterm


def _dx_body(d2_ref, dom_ref, dx_ref):
    d = (d2_ref[0] - d2_ref[1])[:, :16]
    dom = dom_ref[...]
    dx_ref[...] = d - dom * jnp.round(d / dom)


def _edge_body(s2_ref, dx_ref, wx_ref, cm_ref, w2_ref, b2_ref, m_ref):
    pre = s2_ref[0] + s2_ref[1] + _dot(dx_ref[...], wx_ref[...]) + cm_ref[...]
    t = jnp.maximum(pre, 0.0)
    m_ref[...] = jnp.maximum(_dot(t, w2_ref[...]) + b2_ref[...], 0.0)


def _upd_common(h_ref, agg_ref, u1a_ref, u1b_ref, cu_ref, u2_ref, bu2_ref,
                lns_ref, lnb_ref):
    h = h_ref[...]
    agg = agg_ref[0] + agg_ref[1]
    t = jnp.maximum(_dot(h, u1a_ref[...]) + _dot(agg, u1b_ref[...]) + cu_ref[...], 0.0)
    o = h + _dot(t, u2_ref[...]) + bu2_ref[...]
    mu = jnp.mean(o, axis=1, keepdims=True)
    d = o - mu
    var = jnp.mean(d * d, axis=1, keepdims=True)
    return d * lax.rsqrt(var + 1e-5) * lns_ref[...] + lnb_ref[...]


def _upd_mid_body(h_ref, agg_ref, u1a_ref, u1b_ref, cu_ref, u2_ref, bu2_ref,
                  lns_ref, lnb_ref, rv_ref, wd_ref, ws_ref, wv_ref,
                  hn_ref, p_ref, q_ref):
    hn = _upd_common(h_ref, agg_ref, u1a_ref, u1b_ref, cu_ref, u2_ref, bu2_ref,
                     lns_ref, lnb_ref)
    hn_ref[...] = hn
    vterm = _dot(rv_ref[...], wv_ref[...])
    p_ref[...] = _dot(hn, wd_ref[...]) + vterm
    q_ref[...] = _dot(hn, ws_ref[...]) - vterm


def _upd_last_body(h_ref, agg_ref, u1a_ref, u1b_ref, cu_ref, u2_ref, bu2_ref,
                   lns_ref, lnb_ref, hn_ref):
    hn_ref[...] = _upd_common(h_ref, agg_ref, u1a_ref, u1b_ref, cu_ref, u2_ref,
                              bu2_ref, lns_ref, lnb_ref)


def _dec_body(h_ref, o1_ref, bo1_ref, o2_ref, bo2_ref, nf_ref, dom_ref,
              pred_ref, px_ref):
    t = jnp.maximum(_dot(h_ref[...], o1_ref[...]) + bo1_ref[...], 0.0)
    pred = _dot(t, o2_ref[...]) + bo2_ref[...] + nf_ref[...]
    pred_ref[...] = pred
    dom = dom_ref[...]
    px_ref[...] = pred - dom * jnp.floor(pred / dom)


def _pool_body(n_real, bn, h_ref, o_ref):
    i = pl.program_id(0)

    @pl.when(i == 0)
    def _():
        o_ref[...] = jnp.zeros_like(o_ref)

    rows = lax.broadcasted_iota(jnp.int32, (bn, 1), 0) + i * bn
    hm = jnp.where(rows < n_real, h_ref[...], 0.0)
    o_ref[...] += jnp.sum(hm, axis=0, keepdims=True)


def _macro_body(n_real, s_ref, m1_ref, bm1_ref, m2_ref, bm2_ref, o_ref):
    hm = s_ref[...] * (1.0 / n_real)
    t = jnp.maximum(_dot(hm, m1_ref[...]) + bm1_ref[...], 0.0)
    o_ref[...] = _dot(t, m2_ref[...]) + bm2_ref[...]


def kernel(v, pos, r, domain, t, x_global, domain_next, t_next, edge_index,
           batch, params):
    f32 = jnp.float32
    n = v.shape[0]
    e = edge_index.shape[1]
    hid = params['emb2'][0].shape[1]

    npad = -(-n // (_NS * _G)) * (_NS * _G)
    ew = -(-e // (_NW * _G)) * _G
    epad = ew * _NW

    gfeat = jnp.concatenate([domain, t, x_global, domain_next, t_next])

    src = edge_index[0]
    dst = edge_index[1]
    pad_idx = jnp.full((epad - e,), npad - 1, jnp.int32)
    srcp = jnp.concatenate([src, pad_idx])
    dstp = jnp.concatenate([dst, pad_idx])

    def padrows(x, rows, cols):
        return jnp.pad(x, ((0, rows - x.shape[0]), (0, cols - x.shape[1])))

    rv = padrows(jnp.concatenate([r, v], axis=1), npad, 8)
    pos128 = padrows(pos, npad, hid)
    nf16 = padrows(jnp.concatenate([pos, v], axis=1), npad, 16)
    dom16 = jnp.concatenate([domain, jnp.ones((13,), f32)]).reshape(1, 16)
    domn16 = jnp.concatenate([domain_next, jnp.ones((13,), f32)]).reshape(1, 16)
    zeros128 = jnp.zeros((_G, hid), f32)

    e1, be1 = params['emb1']
    e8 = jnp.pad(e1[:7], ((0, 1), (0, 0)))
    ce = (gfeat @ e1[7:19] + be1).reshape(1, hid)
    e2, be2 = params['emb2']
    be2 = be2.reshape(1, hid)

    lw = []
    for lp in params['layers']:
        w1, b1 = lp['msg1']
        wd = w1[0:hid]
        ws = w1[hid:2 * hid]
        wx16 = jnp.pad(w1[2 * hid:2 * hid + 3], ((0, 13), (0, 0)))
        wv8 = jnp.pad(w1[2 * hid + 3:2 * hid + 9], ((1, 1), (0, 0)))
        cm = (gfeat @ w1[2 * hid + 9:] + b1).reshape(1, hid)
        w2, b2 = lp['msg2']
        u1, bu1 = lp['upd1']
        u1a = u1[0:hid]
        u1b = u1[hid:2 * hid]
        cu = (gfeat @ u1[2 * hid:] + bu1).reshape(1, hid)
        u2, bu2 = lp['upd2']
        lw.append(dict(wd=wd, ws=ws, wx16=wx16, wv8=wv8, cm=cm, w2=w2,
                       b2=b2.reshape(1, hid), u1a=u1a, u1b=u1b, cu=cu, u2=u2,
                       bu2=bu2.reshape(1, hid), lns=lp['ln_s'].reshape(1, hid),
                       lnb=lp['ln_b'].reshape(1, hid)))

    o1, bo1 = params['out1']
    o2, bo2 = params['out2']
    o2p = jnp.pad(o2, ((0, 0), (0, 16 - o2.shape[1])))
    bo2p = jnp.pad(bo2, (0, 16 - bo2.shape[0])).reshape(1, 16)
    m1, bm1 = params['mac1']
    m2, bm2 = params['mac2']
    m2p = jnp.pad(m2, ((0, 0), (0, 16 - m2.shape[1])))
    bm2p = jnp.pad(bm2, (0, 16 - bm2.shape[0])).reshape(1, 16)

    bn = 512
    gn = npad // bn
    be = 512
    ge = epad // be

    w128 = _full((hid, hid))
    w1h = _full((1, hid))

    s2_spec = pl.BlockSpec((2, be, hid), lambda i: (0, i, 0))

    enc = pl.pallas_call(
        _enc_body,
        grid=(gn,),
        in_specs=[pl.BlockSpec((bn, 8), lambda i: (i, 0)), _full((8, hid)),
                  w1h, w128, w1h, w128, w128, _full((8, hid))],
        out_specs=[pl.BlockSpec((bn, hid), lambda i: (i, 0))] * 3,
        out_shape=[jax.ShapeDtypeStruct((npad, hid), f32)] * 3,
    )

    dxk = pl.pallas_call(
        _dx_body,
        grid=(ge,),
        in_specs=[s2_spec, _full((1, 16))],
        out_specs=pl.BlockSpec((be, 16), lambda i: (i, 0)),
        out_shape=jax.ShapeDtypeStruct((epad, 16), f32),
    )

    edgek = pl.pallas_call(
        _edge_body,
        grid=(ge,),
        in_specs=[s2_spec,
                  pl.BlockSpec((be, 16), lambda i: (i, 0)),
                  _full((16, hid)), w1h, w128, w1h],
        out_specs=pl.BlockSpec((be, hid), lambda i: (i, 0)),
        out_shape=jax.ShapeDtypeStruct((epad, hid), f32),
    )

    upd_mid = pl.pallas_call(
        _upd_mid_body,
        grid=(gn,),
        in_specs=[pl.BlockSpec((bn, hid), lambda i: (i, 0)),
                  pl.BlockSpec((2, bn, hid), lambda i: (0, i, 0)),
                  w128, w128, w1h, w128, w1h, w1h, w1h,
                  pl.BlockSpec((bn, 8), lambda i: (i, 0)),
                  w128, w128, _full((8, hid))],
        out_specs=[pl.BlockSpec((bn, hid), lambda i: (i, 0))] * 3,
        out_shape=[jax.ShapeDtypeStruct((npad, hid), f32)] * 3,
    )

    upd_last = pl.pallas_call(
        _upd_last_body,
        grid=(gn,),
        in_specs=[pl.BlockSpec((bn, hid), lambda i: (i, 0)),
                  pl.BlockSpec((2, bn, hid), lambda i: (0, i, 0)),
                  w128, w128, w1h, w128, w1h, w1h, w1h],
        out_specs=pl.BlockSpec((bn, hid), lambda i: (i, 0)),
        out_shape=jax.ShapeDtypeStruct((npad, hid), f32),
    )

    deck = pl.pallas_call(
        _dec_body,
        grid=(gn,),
        in_specs=[pl.BlockSpec((bn, hid), lambda i: (i, 0)),
                  w128, w1h, _full((hid, 16)), _full((1, 16)),
                  pl.BlockSpec((bn, 16), lambda i: (i, 0)), _full((1, 16))],
        out_specs=[pl.BlockSpec((bn, 16), lambda i: (i, 0))] * 2,
        out_shape=[jax.ShapeDtypeStruct((npad, 16), f32)] * 2,
    )

    poolk = pl.pallas_call(
        functools.partial(_pool_body, n, bn),
        grid=(gn,),
        in_specs=[pl.BlockSpec((bn, hid), lambda i: (i, 0))],
        out_specs=pl.BlockSpec((1, hid), lambda i: (0, 0)),
        out_shape=jax.ShapeDtypeStruct((1, hid), f32),
    )

    macrok = pl.pallas_call(
        functools.partial(_macro_body, float(n)),
        grid=(1,),
        in_specs=[_full((1, hid)), w128, w1h, _full((hid, 16)), _full((1, 16))],
        out_specs=_full((1, 16)),
        out_shape=jax.ShapeDtypeStruct((1, 16), f32),
    )

    sc_gather = _make_sc_gather(npad, epad, ew, hid)
    sc_scatter = _make_sc_scatter(npad, epad, ew, hid)

    h, p, q = enc(rv, e8, ce, e2, be2, lw[0]['wd'], lw[0]['ws'], lw[0]['wv8'])

    d2 = sc_gather(pos128, pos128, dstp, srcp)
    dxw = dxk(d2, dom16)

    nl = len(lw)
    for l, w in enumerate(lw):
        s2 = sc_gather(p, q, dstp, srcp)
        m = edgek(s2, dxw, w['wx16'], w['cm'], w['w2'], w['b2'])
        agg2 = sc_scatter(m, dstp, zeros128)
        if l + 1 < nl:
            wn = lw[l + 1]
            h, p, q = upd_mid(h, agg2, w['u1a'], w['u1b'], w['cu'], w['u2'],
                              w['bu2'], w['lns'], w['lnb'], rv,
                              wn['wd'], wn['ws'], wn['wv8'])
        else:
            h = upd_last(h, agg2, w['u1a'], w['u1b'], w['cu'], w['u2'],
                         w['bu2'], w['lns'], w['lnb'])

    pred16, px16 = deck(h, o1, bo1.reshape(1, hid), o2p, bo2p, nf16, domn16)
    hsum = poolk(h)
    macro16 = macrok(hsum, m1, bm1.reshape(1, hid), m2p, bm2p)

    pred_x = px16[:n, 0:3]
    pred_v = pred16[:n, 3:9]
    pred_macro = macro16[0, :3]
    return (pred_x, pred_v, pred_macro)
